# baseline (speedup 1.0000x reference)
import jax
import jax.numpy as jnp
from jax import lax
from jax.experimental import pallas as pl
from jax.experimental.pallas import tpu as pltpu
from jax.experimental.pallas import tpu_sc as plsc

_M = 1_000_000
_D = 8
_NW = 32
_CW = 3_968
_NCHUNK = 252
_TAIL0 = _NCHUNK * _CW
_NSLOT = 8
_NBUF = 4
_LAG = _NBUF // 2
_NUPD = 4


def _apply_updates(buf, cw, col0, idx_vec, upd_v, lane16):
    for j in range(_NUPD):
        c_local = idx_vec[j] - col0
        own = (c_local >= 0) & (c_local < cw)

        def _fix(jj=j, c_local=c_local):
            cb = pl.multiple_of((c_local // 16) * 16, 16)
            lo = c_local % 16
            uv = upd_v[pl.ds(jj * 16, 16)]
            for r in range(_D):
                w = buf[r, pl.ds(cb, 16)]
                buf[r, pl.ds(cb, 16)] = w + jnp.where(lane16 == lo, uv[r], 0.0)

        pl.when(own)(_fix)


def _body(x_hbm, upd_hbm, idx_hbm, out_hbm,
          b0, b1, b2, b3, rsems, wsems, upd_v, idx_v):
    wid = lax.axis_index("s") * 2 + lax.axis_index("c")
    bufs = (b0, b1, b2, b3)

    pltpu.sync_copy(upd_hbm, upd_v)
    pltpu.sync_copy(idx_hbm, idx_v)
    idx_vec = idx_v[...]
    lane16 = lax.iota(jnp.int32, 16)

    def chunk_id(k):
        return wid + k * _NW

    def valid(k):
        return chunk_id(k) < _NCHUNK

    def rd(k):
        b = k % _NBUF
        col0 = chunk_id(k) * _CW
        return pltpu.make_async_copy(
            x_hbm.at[:, pl.ds(col0, _CW)], bufs[b], rsems.at[b])

    def wr(k):
        b = k % _NBUF
        col0 = chunk_id(k) * _CW
        return pltpu.make_async_copy(
            bufs[b], out_hbm.at[:, pl.ds(col0, _CW)], wsems.at[b])

    for k in range(_LAG):
        pl.when(valid(k))(lambda k=k: rd(k).start())
    for k in range(_NSLOT):
        if k >= _LAG:
            pl.when(valid(k - _LAG))(lambda k=k: wr(k - _LAG).wait())
        if k + _LAG < _NSLOT:
            pl.when(valid(k + _LAG))(lambda k=k: rd(k + _LAG).start())

        def _proc(k=k):
            rd(k).wait()
            col0 = chunk_id(k) * _CW
            _apply_updates(bufs[k % _NBUF], _CW, col0, idx_vec, upd_v, lane16)
            wr(k).start()

        pl.when(valid(k))(_proc)
    for k in range(_NSLOT - _LAG, _NSLOT):
        pl.when(valid(k))(lambda k=k: wr(k).wait())

    @pl.when(wid == _NW - 1)
    def _tail():
        toff = pl.multiple_of(_TAIL0 + 0 * wid, 128)
        tb = b0.at[:, pl.ds(0, 128)]
        pltpu.make_async_copy(x_hbm.at[:, pl.ds(toff, 128)], tb, rsems.at[0]).start()
        pltpu.make_async_copy(x_hbm.at[:, pl.ds(toff, 128)], tb, rsems.at[0]).wait()
        _apply_updates(b0, 128, _TAIL0, idx_vec, upd_v, lane16)
        pltpu.make_async_copy(tb, out_hbm.at[:, pl.ds(toff, 128)], wsems.at[0]).start()
        pltpu.make_async_copy(tb, out_hbm.at[:, pl.ds(toff, 128)], wsems.at[0]).wait()


def kernel(x, update, index):
    xt = x.T
    upd_pad = jnp.zeros((_NUPD, 16), jnp.float32).at[:, :_D].set(update).reshape(-1)
    idx_pad = jnp.zeros((16,), jnp.int32).at[:_NUPD].set(index)

    mesh = plsc.VectorSubcoreMesh(
        core_axis_name="c", subcore_axis_name="s", num_cores=2, num_subcores=16
    )
    out = pl.kernel(
        _body,
        out_type=jax.ShapeDtypeStruct((_D, _M), jnp.float32),
        mesh=mesh,
        scratch_types=[
            pltpu.VMEM((_D, _CW), jnp.float32),
            pltpu.VMEM((_D, _CW), jnp.float32),
            pltpu.VMEM((_D, _CW), jnp.float32),
            pltpu.VMEM((_D, _CW), jnp.float32),
            pltpu.SemaphoreType.DMA((_NBUF,)),
            pltpu.SemaphoreType.DMA((_NBUF,)),
            pltpu.VMEM((_NUPD * 16,), jnp.float32),
            pltpu.VMEM((16,), jnp.int32),
        ],
    )(xt, upd_pad, idx_pad)
    return out.T
